# baseline (speedup 1.0000x reference)
import functools

import jax
import jax.numpy as jnp
from jax import lax
from jax.experimental import pallas as pl
from jax.experimental.pallas import tpu as pltpu
from jax.experimental.pallas import tpu_sc as plsc

NN = 10000
NE = 320000
DI, DH, DO = 128, 64, 32
NC, NS = 2, 16
NW = NC * NS
EPT = NE // NW

_MESH = plsc.VectorSubcoreMesh(
    core_axis_name="c", subcore_axis_name="s", num_cores=NC, num_subcores=NS)
_SC_PARAMS = pltpu.CompilerParams(use_tc_tiling_on_sc=False,
                                  needs_layout_passes=False)

_f32 = jnp.float32
_i32 = jnp.int32


_DEG_B = 2000

def _deg_body(dst_hbm, degp_hbm, hist_sh, idx_v, ones_v, buf_v):
    cid = lax.axis_index("c")
    sid = lax.axis_index("s")
    wid = cid * NS + sid
    for i in range(_DEG_B // 16):
        ones_v[pl.ds(16 * i, 16)] = jnp.full((16,), 1.0, _f32)
    for i in range(1024 // 16):
        buf_v[pl.ds(16 * i, 16)] = jnp.zeros((16,), _f32)

    @pl.when(sid < 10)
    def _():
        pltpu.sync_copy(buf_v.at[pl.ds(0, 1000)],
                        hist_sh.at[pl.ds(sid * 1000, 1000)])

    plsc.subcore_barrier()
    for it in range(EPT // _DEG_B):
        base = wid * EPT + it * _DEG_B
        pltpu.sync_copy(dst_hbm.at[pl.ds(base, _DEG_B)], idx_v)
        pltpu.sync_copy(ones_v, hist_sh.at[idx_v], add=True)
    plsc.subcore_barrier()

    @pl.when(sid < 10)
    def _():
        pltpu.sync_copy(hist_sh.at[pl.ds(sid * 1000, 1000)],
                        buf_v.at[pl.ds(0, 1000)])
        pltpu.sync_copy(buf_v.at[pl.ds(0, 1000)],
                        degp_hbm.at[pl.ds(cid * NN + sid * 1000, 1000)])


_deg_call = pl.kernel(
    _deg_body,
    out_type=jax.ShapeDtypeStruct((NC * NN,), _f32),
    mesh=_MESH,
    compiler_params=_SC_PARAMS,
    scratch_types=[
        pltpu.VMEM_SHARED((NN,), _f32),
        pltpu.VMEM((_DEG_B,), _i32),
        pltpu.VMEM((_DEG_B,), _f32),
        pltpu.VMEM((1024,), _f32),
    ],
)


def _make_agg(D):
    EB = 200 if D > 32 else 400
    NB = EPT // EB
    NBUF = 5
    RZ = 40

    def body(src_hbm, dst_hbm, tab_hbm, pp_hbm,
             acc_sh, sidx_v, didx_v, rows, gsem, ssem, zb_v, wsem):
        cid = lax.axis_index("c")
        sid = lax.axis_index("s")
        wid = cid * NS + sid
        tbase = wid * EPT
        pltpu.sync_copy(src_hbm.at[pl.ds(tbase, EPT)], sidx_v)
        pltpu.sync_copy(dst_hbm.at[pl.ds(tbase, EPT)], didx_v)
        for r in range(RZ):
            for c in range(D // 16):
                zb_v[r, pl.ds(16 * c, 16)] = jnp.zeros((16,), _f32)

        @pl.when(sid < 10)
        def _():
            for k in range(1000 // RZ):
                pltpu.sync_copy(zb_v, acc_sh.at[pl.ds(sid * 1000 + k * RZ, RZ)])

        plsc.subcore_barrier()

        def sidx(j):
            return sidx_v.at[pl.ds(j * EB, EB)]

        def didx(j):
            return didx_v.at[pl.ds(j * EB, EB)]

        LAG = 3
        gd = [None] * NB
        sd = [None] * NB
        for j in range(NB):
            p = j % NBUF
            if j >= NBUF:
                sd[j - NBUF].wait()
            gd[j] = pltpu.async_copy(tab_hbm.at[sidx(j)], rows[p], gsem[p])
            if j >= LAG:
                q = (j - LAG) % NBUF
                gd[j - LAG].wait()
                sd[j - LAG] = pltpu.async_copy(rows[q],
                                               acc_sh.at[didx(j - LAG)],
                                               ssem[q], add=True)
        for j in range(NB - LAG, NB):
            q = j % NBUF
            gd[j].wait()
            sd[j] = pltpu.async_copy(rows[q], acc_sh.at[didx(j)],
                                     ssem[q], add=True)
        for t in range(NB - NBUF, NB):
            sd[t].wait()
        plsc.subcore_barrier()

        @pl.when(sid < 10)
        def _():
            for k in range(5):
                r0 = sid * 1000 + k * 200
                pltpu.sync_copy(acc_sh.at[pl.ds(r0, 200)],
                                rows[0].at[pl.ds(0, 200)])
                pltpu.sync_copy(rows[0].at[pl.ds(0, 200)],
                                pp_hbm.at[pl.ds(cid * NN + r0, 200)])

    return pl.kernel(
        body,
        out_type=jax.ShapeDtypeStruct((NC * NN, D), _f32),
        mesh=_MESH,
        compiler_params=_SC_PARAMS,
        scratch_types=[
            pltpu.VMEM_SHARED((NN, D), _f32),
            pltpu.VMEM((EPT,), _i32),
            pltpu.VMEM((EPT,), _i32),
            [pltpu.VMEM((EB, D), _f32)] * NBUF,
            [pltpu.SemaphoreType.DMA] * NBUF,
            [pltpu.SemaphoreType.DMA] * NBUF,
            pltpu.VMEM((RZ, D), _f32),
            pltpu.SemaphoreType.DMA,
        ],
    )


_agg_h = _make_agg(DH)
_agg_o = _make_agg(DO)


_DEC_B = 400

_DEC_B = 400
_DEC_NB = EPT // _DEC_B

def _dec_body(src_hbm, dst_hbm, qp_hbm, h2s_hbm, dinv_hbm, b2_hbm,
              out_hbm, zcp_hbm,
              sidx_v, didx_v, zs, zd, ov, gsem, osem, s_in, s_out):
    cid = lax.axis_index("c")
    sid = lax.axis_index("s")
    wid = cid * NS + sid
    tbase = wid * EPT
    lanes = lax.iota(_i32, 16)
    pltpu.sync_copy(src_hbm.at[pl.ds(tbase, EPT)], sidx_v)
    pltpu.sync_copy(dst_hbm.at[pl.ds(tbase, EPT)], didx_v)

    pltpu.sync_copy(b2_hbm, ov[1].at[pl.ds(0, DO)])
    b2lo = ov[1][pl.ds(0, 16)]
    b2hi = ov[1][pl.ds(16, 16)]

    @pl.when(sid < 10)
    def _():
        def start_stage(c):
            h = (c % 2) * 200
            r0 = sid * 1000 + c * 200
            b = 4 * (c % 2)
            return (
                pltpu.async_copy(qp_hbm.at[pl.ds(r0, 200)],
                                 zs[0].at[pl.ds(h, 200)], s_in[b]),
                pltpu.async_copy(qp_hbm.at[pl.ds(NN + r0, 200)],
                                 zs[1].at[pl.ds(h, 200)], s_in[b + 1]),
                pltpu.async_copy(h2s_hbm.at[pl.ds(r0, 200)],
                                 zs[2].at[pl.ds(h, 200)], s_in[b + 2]),
                pltpu.async_copy(dinv_hbm.at[pl.ds(r0, 200)],
                                 ov[0].at[pl.ds(h, 200)], s_in[b + 3]),
            )

        ind = [None] * 5
        outd = [None] * 5
        ind[0] = start_stage(0)
        for c in range(5):
            h = (c % 2) * 200
            r0 = sid * 1000 + c * 200
            if c + 1 < 5:
                ind[c + 1] = start_stage(c + 1)
            for dsc in ind[c]:
                dsc.wait()
            if c >= 2:
                outd[c - 2].wait()

            def zrow(i, carry):
                dv = ov[0][pl.ds(h + i, 16)][0]
                lo = (zs[0][h + i, pl.ds(0, 16)] + zs[1][h + i, pl.ds(0, 16)]
                      + zs[2][h + i, pl.ds(0, 16)]) * dv + b2lo
                hi = (zs[0][h + i, pl.ds(16, 16)] + zs[1][h + i, pl.ds(16, 16)]
                      + zs[2][h + i, pl.ds(16, 16)]) * dv + b2hi
                zd[0][h + i, pl.ds(0, 16)] = lo
                zd[0][h + i, pl.ds(16, 16)] = hi
                return carry

            lax.fori_loop(0, 200, zrow, 0)
            outd[c] = pltpu.async_copy(zd[0].at[pl.ds(h, 200)],
                                       zcp_hbm.at[pl.ds(cid * NN + r0, 200)],
                                       s_out[c % 2])
        outd[3].wait()
        outd[4].wait()

    def bias_idx(i, carry):
        o = 16 * i
        sidx_v[pl.ds(o, 16)] = sidx_v[pl.ds(o, 16)] + cid * NN
        didx_v[pl.ds(o, 16)] = didx_v[pl.ds(o, 16)] + cid * NN
        return carry

    lax.fori_loop(0, EPT // 16, bias_idx, 0)
    plsc.subcore_barrier()

    def compute(zs_v, zd_v, ov_v):
        def group(g, carry):
            rows = g * 16 + lanes
            zs_g = zs_v.at[pl.ds(g * 16, 16)]
            zd_g = zd_v.at[pl.ds(g * 16, 16)]
            accs = [jnp.zeros((16,), _f32) for _ in range(4)]
            for d in range(DO):
                col = (lanes + d) & (DO - 1)
                a = plsc.load_gather(zs_g, [lanes, col])
                b = plsc.load_gather(zd_g, [lanes, col])
                accs[d % 4] = accs[d % 4] + a * b
            acc = (accs[0] + accs[1]) + (accs[2] + accs[3])
            sg = 1.0 / (1.0 + jnp.exp(-acc))
            plsc.store_scatter(ov_v, [rows], sg)
            return carry
        lax.fori_loop(0, _DEC_B // 16, group, 0)

    def start_gather(k):
        p = k % 3
        e0 = k * _DEC_B
        src = zcp_hbm
        return (
            pltpu.async_copy(src.at[sidx_v.at[pl.ds(e0, _DEC_B)]],
                             zs[p], gsem[2 * p]),
            pltpu.async_copy(src.at[didx_v.at[pl.ds(e0, _DEC_B)]],
                             zd[p], gsem[2 * p + 1]),
        )

    gd = [None] * _DEC_NB
    od = [None] * _DEC_NB
    gd[0] = start_gather(0)
    gd[1] = start_gather(1)
    for k in range(_DEC_NB):
        p = k % 3
        if k + 2 < _DEC_NB:
            gd[k + 2] = start_gather(k + 2)
        gd[k][0].wait()
        gd[k][1].wait()
        if k >= 2:
            od[k - 2].wait()
        q = k % 2
        compute(zs[p], zd[p], ov[q])
        od[k] = pltpu.async_copy(ov[q].at[pl.ds(0, _DEC_B)],
                                 out_hbm.at[pl.ds(tbase + k * _DEC_B,
                                                  _DEC_B)], osem[q])
    od[_DEC_NB - 2].wait()
    od[_DEC_NB - 1].wait()


_dec_call = pl.kernel(
    _dec_body,
    out_type=(jax.ShapeDtypeStruct((NE,), _f32),
              jax.ShapeDtypeStruct((NC * NN, DO), _f32)),
    mesh=_MESH,
    compiler_params=_SC_PARAMS,
    scratch_types=[
        pltpu.VMEM((EPT,), _i32),
        pltpu.VMEM((EPT,), _i32),
        [pltpu.VMEM((_DEC_B, DO), _f32)] * 3,
        [pltpu.VMEM((_DEC_B, DO), _f32)] * 3,
        [pltpu.VMEM((_DEC_B + 16,), _f32)] * 2,
        [pltpu.SemaphoreType.DMA] * 6,
        [pltpu.SemaphoreType.DMA] * 2,
        [pltpu.SemaphoreType.DMA] * 8,
        [pltpu.SemaphoreType.DMA] * 2,
    ],
)


def _enc1_body(degp_ref, x_ref, w1_ref, dinv_ref, hs_ref):
    deg = degp_ref[:NN] + degp_ref[NN:] + 1.0
    dinv = lax.rsqrt(deg)
    dinv_ref[...] = dinv
    h = jnp.dot(x_ref[...], w1_ref[...], preferred_element_type=_f32)
    hs_ref[...] = h * dinv[:, None]


_enc1_call = pl.pallas_call(
    _enc1_body,
    out_shape=(jax.ShapeDtypeStruct((NN,), _f32),
               jax.ShapeDtypeStruct((NN, DH), _f32)),
)


def _enc2_body(p_ref, hs_ref, dinv_ref, b1_ref, w2_ref, h2s_ref):
    dinv = dinv_ref[...]
    h = jnp.maximum(
        dinv[:, None] * (p_ref[:NN, :] + p_ref[NN:, :] + hs_ref[...])
        + b1_ref[...], 0.0)
    h2 = jnp.dot(h, w2_ref[...], preferred_element_type=_f32)
    h2s_ref[...] = h2 * dinv[:, None]


_enc2_call = pl.pallas_call(
    _enc2_body,
    out_shape=jax.ShapeDtypeStruct((NN, DO), _f32),
)


def kernel(x, edge_index, W1, b1, W2, b2):
    src = edge_index[0].astype(_i32)
    dst = edge_index[1].astype(_i32)
    degp = _deg_call(dst)
    dinv, hs = _enc1_call(degp, x, W1)
    P = _agg_h(src, dst, hs)
    h2s = _enc2_call(P, hs, dinv, b1, W2)
    Q = _agg_o(src, dst, h2s)
    out, _ = _dec_call(src, dst, Q, h2s, dinv, b2)
    return out


# speedup vs baseline: 1.1140x; 1.1140x over previous
import functools

import jax
import jax.numpy as jnp
from jax import lax
from jax.experimental import pallas as pl
from jax.experimental.pallas import tpu as pltpu
from jax.experimental.pallas import tpu_sc as plsc

NN = 10000
NE = 320000
DI, DH, DO = 128, 64, 32
NC, NS = 2, 16
NW = NC * NS
EPT = NE // NW

_MESH = plsc.VectorSubcoreMesh(
    core_axis_name="c", subcore_axis_name="s", num_cores=NC, num_subcores=NS)
_SC_PARAMS = pltpu.CompilerParams(use_tc_tiling_on_sc=False,
                                  needs_layout_passes=False)

_f32 = jnp.float32
_i32 = jnp.int32


_DEG_B = 2000

def _deg_body(dst_hbm, degp_hbm, hist_sh, idx_v, ones_v, buf_v):
    cid = lax.axis_index("c")
    sid = lax.axis_index("s")
    wid = cid * NS + sid
    for i in range(_DEG_B // 16):
        ones_v[pl.ds(16 * i, 16)] = jnp.full((16,), 1.0, _f32)
    for i in range(1024 // 16):
        buf_v[pl.ds(16 * i, 16)] = jnp.zeros((16,), _f32)

    @pl.when(sid < 10)
    def _():
        pltpu.sync_copy(buf_v.at[pl.ds(0, 1000)],
                        hist_sh.at[pl.ds(sid * 1000, 1000)])

    plsc.subcore_barrier()
    for it in range(EPT // _DEG_B):
        base = wid * EPT + it * _DEG_B
        pltpu.sync_copy(dst_hbm.at[pl.ds(base, _DEG_B)], idx_v)
        pltpu.sync_copy(ones_v, hist_sh.at[idx_v], add=True)
    plsc.subcore_barrier()

    @pl.when(sid < 10)
    def _():
        pltpu.sync_copy(hist_sh.at[pl.ds(sid * 1000, 1000)],
                        buf_v.at[pl.ds(0, 1000)])
        pltpu.sync_copy(buf_v.at[pl.ds(0, 1000)],
                        degp_hbm.at[pl.ds(cid * NN + sid * 1000, 1000)])


_deg_call = pl.kernel(
    _deg_body,
    out_type=jax.ShapeDtypeStruct((NC * NN,), _f32),
    mesh=_MESH,
    compiler_params=_SC_PARAMS,
    scratch_types=[
        pltpu.VMEM_SHARED((NN,), _f32),
        pltpu.VMEM((_DEG_B,), _i32),
        pltpu.VMEM((_DEG_B,), _f32),
        pltpu.VMEM((1024,), _f32),
    ],
)


def _make_agg(D):
    EB = 200 if D > 32 else 400
    NB = EPT // EB
    NBUF = 5
    RZ = 40

    def body(src_hbm, dst_hbm, tab_hbm, pp_hbm,
             acc_sh, sidx_v, didx_v, rows, gsem, ssem, zb_v, wsem):
        cid = lax.axis_index("c")
        sid = lax.axis_index("s")
        wid = cid * NS + sid
        tbase = wid * EPT
        pltpu.sync_copy(src_hbm.at[pl.ds(tbase, EPT)], sidx_v)
        pltpu.sync_copy(dst_hbm.at[pl.ds(tbase, EPT)], didx_v)
        for r in range(RZ):
            for c in range(D // 16):
                zb_v[r, pl.ds(16 * c, 16)] = jnp.zeros((16,), _f32)

        @pl.when(sid < 10)
        def _():
            for k in range(1000 // RZ):
                pltpu.sync_copy(zb_v, acc_sh.at[pl.ds(sid * 1000 + k * RZ, RZ)])

        plsc.subcore_barrier()

        def sidx(j):
            return sidx_v.at[pl.ds(j * EB, EB)]

        def didx(j):
            return didx_v.at[pl.ds(j * EB, EB)]

        LAG = 3
        gd = [None] * NB
        sd = [None] * NB
        for j in range(NB):
            p = j % NBUF
            if j >= NBUF:
                sd[j - NBUF].wait()
            gd[j] = pltpu.async_copy(tab_hbm.at[sidx(j)], rows[p], gsem[p])
            if j >= LAG:
                q = (j - LAG) % NBUF
                gd[j - LAG].wait()
                sd[j - LAG] = pltpu.async_copy(rows[q],
                                               acc_sh.at[didx(j - LAG)],
                                               ssem[q], add=True)
        for j in range(NB - LAG, NB):
            q = j % NBUF
            gd[j].wait()
            sd[j] = pltpu.async_copy(rows[q], acc_sh.at[didx(j)],
                                     ssem[q], add=True)
        for t in range(NB - NBUF, NB):
            sd[t].wait()
        plsc.subcore_barrier()

        @pl.when(sid < 10)
        def _():
            for k in range(5):
                r0 = sid * 1000 + k * 200
                pltpu.sync_copy(acc_sh.at[pl.ds(r0, 200)],
                                rows[0].at[pl.ds(0, 200)])
                pltpu.sync_copy(rows[0].at[pl.ds(0, 200)],
                                pp_hbm.at[pl.ds(cid * NN + r0, 200)])

    return pl.kernel(
        body,
        out_type=jax.ShapeDtypeStruct((NC * NN, D), _f32),
        mesh=_MESH,
        compiler_params=_SC_PARAMS,
        scratch_types=[
            pltpu.VMEM_SHARED((NN, D), _f32),
            pltpu.VMEM((EPT,), _i32),
            pltpu.VMEM((EPT,), _i32),
            [pltpu.VMEM((EB, D), _f32)] * NBUF,
            [pltpu.SemaphoreType.DMA] * NBUF,
            [pltpu.SemaphoreType.DMA] * NBUF,
            pltpu.VMEM((RZ, D), _f32),
            pltpu.SemaphoreType.DMA,
        ],
    )


_agg_h = _make_agg(DH)
_agg_o = _make_agg(DO)


_DEC_B = 400

_DEC_B = 400
_DEC_NB = EPT // _DEC_B

def _dec_body(src_hbm, dst_hbm, qp_hbm, h2s_hbm, dinv_hbm, b2_hbm,
              out_hbm, zcp_hbm,
              sidx_v, didx_v, zs, zd, ov, gsem, osem, s_in, s_out):
    cid = lax.axis_index("c")
    sid = lax.axis_index("s")
    wid = cid * NS + sid
    tbase = wid * EPT
    lanes = lax.iota(_i32, 16)
    pltpu.sync_copy(src_hbm.at[pl.ds(tbase, EPT)], sidx_v)
    pltpu.sync_copy(dst_hbm.at[pl.ds(tbase, EPT)], didx_v)

    pltpu.sync_copy(b2_hbm, ov[1].at[pl.ds(0, DO)])
    b2lo = ov[1][pl.ds(0, 16)]
    b2hi = ov[1][pl.ds(16, 16)]

    @pl.when(sid < 10)
    def _():
        def start_stage(c):
            h = (c % 2) * 200
            r0 = sid * 1000 + c * 200
            b = 4 * (c % 2)
            return (
                pltpu.async_copy(qp_hbm.at[pl.ds(r0, 200)],
                                 zs[0].at[pl.ds(h, 200)], s_in[b]),
                pltpu.async_copy(qp_hbm.at[pl.ds(NN + r0, 200)],
                                 zs[1].at[pl.ds(h, 200)], s_in[b + 1]),
                pltpu.async_copy(h2s_hbm.at[pl.ds(r0, 200)],
                                 zs[2].at[pl.ds(h, 200)], s_in[b + 2]),
                pltpu.async_copy(dinv_hbm.at[pl.ds(r0, 200)],
                                 ov[0].at[pl.ds(h, 200)], s_in[b + 3]),
            )

        ind = [None] * 5
        outd = [None] * 5
        ind[0] = start_stage(0)
        for c in range(5):
            h = (c % 2) * 200
            r0 = sid * 1000 + c * 200
            if c + 1 < 5:
                ind[c + 1] = start_stage(c + 1)
            for dsc in ind[c]:
                dsc.wait()
            if c >= 2:
                outd[c - 2].wait()

            def zrow(i, carry):
                dv = ov[0][pl.ds(h + i, 16)][0]
                lo = (zs[0][h + i, pl.ds(0, 16)] + zs[1][h + i, pl.ds(0, 16)]
                      + zs[2][h + i, pl.ds(0, 16)]) * dv + b2lo
                hi = (zs[0][h + i, pl.ds(16, 16)] + zs[1][h + i, pl.ds(16, 16)]
                      + zs[2][h + i, pl.ds(16, 16)]) * dv + b2hi
                zd[0][h + i, pl.ds(0, 16)] = lo
                zd[0][h + i, pl.ds(16, 16)] = hi
                return carry

            lax.fori_loop(0, 200, zrow, 0)
            outd[c] = pltpu.async_copy(zd[0].at[pl.ds(h, 200)],
                                       zcp_hbm.at[pl.ds(cid * NN + r0, 200)],
                                       s_out[c % 2])
        outd[3].wait()
        outd[4].wait()

    def bias_idx(i, carry):
        o = 16 * i
        sidx_v[pl.ds(o, 16)] = sidx_v[pl.ds(o, 16)] + cid * NN
        didx_v[pl.ds(o, 16)] = didx_v[pl.ds(o, 16)] + cid * NN
        return carry

    lax.fori_loop(0, EPT // 16, bias_idx, 0)
    plsc.subcore_barrier()

    def compute(zs_v, zd_v, ov_v):
        def group(g, carry):
            rows = g * 16 + lanes
            accs = [jnp.zeros((16,), _f32) for _ in range(4)]
            for d in range(DO):
                col = (lanes + d) & (DO - 1)
                a = plsc.load_gather(zs_v, [rows, col])
                b = plsc.load_gather(zd_v, [rows, col])
                accs[d % 4] = accs[d % 4] + a * b
            acc = (accs[0] + accs[1]) + (accs[2] + accs[3])
            sg = 1.0 / (1.0 + jnp.exp(-acc))
            plsc.store_scatter(ov_v, [rows], sg)
            return carry
        lax.fori_loop(0, _DEC_B // 16, group, 0)

    def start_gather(k):
        p = k % 3
        e0 = k * _DEC_B
        src = zcp_hbm
        return (
            pltpu.async_copy(src.at[sidx_v.at[pl.ds(e0, _DEC_B)]],
                             zs[p], gsem[2 * p]),
            pltpu.async_copy(src.at[didx_v.at[pl.ds(e0, _DEC_B)]],
                             zd[p], gsem[2 * p + 1]),
        )

    gd = [None] * _DEC_NB
    od = [None] * _DEC_NB
    gd[0] = start_gather(0)
    gd[1] = start_gather(1)
    for k in range(_DEC_NB):
        p = k % 3
        if k + 2 < _DEC_NB:
            gd[k + 2] = start_gather(k + 2)
        gd[k][0].wait()
        gd[k][1].wait()
        if k >= 2:
            od[k - 2].wait()
        q = k % 2
        compute(zs[p], zd[p], ov[q])
        od[k] = pltpu.async_copy(ov[q].at[pl.ds(0, _DEC_B)],
                                 out_hbm.at[pl.ds(tbase + k * _DEC_B,
                                                  _DEC_B)], osem[q])
    od[_DEC_NB - 2].wait()
    od[_DEC_NB - 1].wait()


_dec_call = pl.kernel(
    _dec_body,
    out_type=(jax.ShapeDtypeStruct((NE,), _f32),
              jax.ShapeDtypeStruct((NC * NN, DO), _f32)),
    mesh=_MESH,
    compiler_params=_SC_PARAMS,
    scratch_types=[
        pltpu.VMEM((EPT,), _i32),
        pltpu.VMEM((EPT,), _i32),
        [pltpu.VMEM((_DEC_B, DO), _f32)] * 3,
        [pltpu.VMEM((_DEC_B, DO), _f32)] * 3,
        [pltpu.VMEM((_DEC_B + 16,), _f32)] * 2,
        [pltpu.SemaphoreType.DMA] * 6,
        [pltpu.SemaphoreType.DMA] * 2,
        [pltpu.SemaphoreType.DMA] * 8,
        [pltpu.SemaphoreType.DMA] * 2,
    ],
)


def _enc1_body(degp_ref, x_ref, w1_ref, dinv_ref, hs_ref):
    deg = degp_ref[:NN] + degp_ref[NN:] + 1.0
    dinv = lax.rsqrt(deg)
    dinv_ref[...] = dinv
    h = jnp.dot(x_ref[...], w1_ref[...], preferred_element_type=_f32)
    hs_ref[...] = h * dinv[:, None]


_enc1_call = pl.pallas_call(
    _enc1_body,
    out_shape=(jax.ShapeDtypeStruct((NN,), _f32),
               jax.ShapeDtypeStruct((NN, DH), _f32)),
)


def _enc2_body(p_ref, hs_ref, dinv_ref, b1_ref, w2_ref, h2s_ref):
    dinv = dinv_ref[...]
    h = jnp.maximum(
        dinv[:, None] * (p_ref[:NN, :] + p_ref[NN:, :] + hs_ref[...])
        + b1_ref[...], 0.0)
    h2 = jnp.dot(h, w2_ref[...], preferred_element_type=_f32)
    h2s_ref[...] = h2 * dinv[:, None]


_enc2_call = pl.pallas_call(
    _enc2_body,
    out_shape=jax.ShapeDtypeStruct((NN, DO), _f32),
)


def kernel(x, edge_index, W1, b1, W2, b2):
    src = edge_index[0].astype(_i32)
    dst = edge_index[1].astype(_i32)
    degp = _deg_call(dst)
    dinv, hs = _enc1_call(degp, x, W1)
    P = _agg_h(src, dst, hs)
    h2s = _enc2_call(P, hs, dinv, b1, W2)
    Q = _agg_o(src, dst, h2s)
    out, _ = _dec_call(src, dst, Q, h2s, dinv, b2)
    return out


# speedup vs baseline: 1.1144x; 1.0004x over previous
import functools

import jax
import jax.numpy as jnp
from jax import lax
from jax.experimental import pallas as pl
from jax.experimental.pallas import tpu as pltpu
from jax.experimental.pallas import tpu_sc as plsc

NN = 10000
NE = 320000
DI, DH, DO = 128, 64, 32
NC, NS = 2, 16
NW = NC * NS
EPT = NE // NW

_MESH = plsc.VectorSubcoreMesh(
    core_axis_name="c", subcore_axis_name="s", num_cores=NC, num_subcores=NS)
_SC_PARAMS = pltpu.CompilerParams(use_tc_tiling_on_sc=False,
                                  needs_layout_passes=False)

_f32 = jnp.float32
_i32 = jnp.int32


_DEG_B = 2000

def _deg_body(dst_hbm, degp_hbm, hist_sh, idx_v, ones_v, buf_v):
    cid = lax.axis_index("c")
    sid = lax.axis_index("s")
    wid = cid * NS + sid
    for i in range(_DEG_B // 16):
        ones_v[pl.ds(16 * i, 16)] = jnp.full((16,), 1.0, _f32)
    for i in range(1024 // 16):
        buf_v[pl.ds(16 * i, 16)] = jnp.zeros((16,), _f32)

    @pl.when(sid < 10)
    def _():
        pltpu.sync_copy(buf_v.at[pl.ds(0, 1000)],
                        hist_sh.at[pl.ds(sid * 1000, 1000)])

    plsc.subcore_barrier()
    for it in range(EPT // _DEG_B):
        base = wid * EPT + it * _DEG_B
        pltpu.sync_copy(dst_hbm.at[pl.ds(base, _DEG_B)], idx_v)
        pltpu.sync_copy(ones_v, hist_sh.at[idx_v], add=True)
    plsc.subcore_barrier()

    @pl.when(sid < 10)
    def _():
        pltpu.sync_copy(hist_sh.at[pl.ds(sid * 1000, 1000)],
                        buf_v.at[pl.ds(0, 1000)])
        pltpu.sync_copy(buf_v.at[pl.ds(0, 1000)],
                        degp_hbm.at[pl.ds(cid * NN + sid * 1000, 1000)])


_deg_call = pl.kernel(
    _deg_body,
    out_type=jax.ShapeDtypeStruct((NC * NN,), _f32),
    mesh=_MESH,
    compiler_params=_SC_PARAMS,
    scratch_types=[
        pltpu.VMEM_SHARED((NN,), _f32),
        pltpu.VMEM((_DEG_B,), _i32),
        pltpu.VMEM((_DEG_B,), _f32),
        pltpu.VMEM((1024,), _f32),
    ],
)


def _make_agg(D):
    EB = 200 if D > 32 else 400
    NB = EPT // EB
    NBUF = 5
    RZ = 40

    def body(src_hbm, dst_hbm, tab_hbm, pp_hbm,
             acc_sh, sidx_v, didx_v, rows, gsem, ssem, zb_v, wsem):
        cid = lax.axis_index("c")
        sid = lax.axis_index("s")
        wid = cid * NS + sid
        tbase = wid * EPT
        pltpu.sync_copy(src_hbm.at[pl.ds(tbase, EPT)], sidx_v)
        pltpu.sync_copy(dst_hbm.at[pl.ds(tbase, EPT)], didx_v)
        for r in range(RZ):
            for c in range(D // 16):
                zb_v[r, pl.ds(16 * c, 16)] = jnp.zeros((16,), _f32)

        @pl.when(sid < 10)
        def _():
            for k in range(1000 // RZ):
                pltpu.sync_copy(zb_v, acc_sh.at[pl.ds(sid * 1000 + k * RZ, RZ)])

        plsc.subcore_barrier()

        def sidx(j):
            return sidx_v.at[pl.ds(j * EB, EB)]

        def didx(j):
            return didx_v.at[pl.ds(j * EB, EB)]

        LAG = 3
        gd = [None] * NB
        sd = [None] * NB
        for j in range(NB):
            p = j % NBUF
            if j >= NBUF:
                sd[j - NBUF].wait()
            gd[j] = pltpu.async_copy(tab_hbm.at[sidx(j)], rows[p], gsem[p])
            if j >= LAG:
                q = (j - LAG) % NBUF
                gd[j - LAG].wait()
                sd[j - LAG] = pltpu.async_copy(rows[q],
                                               acc_sh.at[didx(j - LAG)],
                                               ssem[q], add=True)
        for j in range(NB - LAG, NB):
            q = j % NBUF
            gd[j].wait()
            sd[j] = pltpu.async_copy(rows[q], acc_sh.at[didx(j)],
                                     ssem[q], add=True)
        for t in range(NB - NBUF, NB):
            sd[t].wait()
        plsc.subcore_barrier()

        @pl.when(sid < 10)
        def _():
            for k in range(5):
                r0 = sid * 1000 + k * 200
                pltpu.sync_copy(acc_sh.at[pl.ds(r0, 200)],
                                rows[0].at[pl.ds(0, 200)])
                pltpu.sync_copy(rows[0].at[pl.ds(0, 200)],
                                pp_hbm.at[pl.ds(cid * NN + r0, 200)])

    return pl.kernel(
        body,
        out_type=jax.ShapeDtypeStruct((NC * NN, D), _f32),
        mesh=_MESH,
        compiler_params=_SC_PARAMS,
        scratch_types=[
            pltpu.VMEM_SHARED((NN, D), _f32),
            pltpu.VMEM((EPT,), _i32),
            pltpu.VMEM((EPT,), _i32),
            [pltpu.VMEM((EB, D), _f32)] * NBUF,
            [pltpu.SemaphoreType.DMA] * NBUF,
            [pltpu.SemaphoreType.DMA] * NBUF,
            pltpu.VMEM((RZ, D), _f32),
            pltpu.SemaphoreType.DMA,
        ],
    )


_agg_h = _make_agg(DH)
_agg_o = _make_agg(DO)


_DEC_B = 400

_DEC_B = 400
_DEC_NB = EPT // _DEC_B

def _dec_body(src_hbm, dst_hbm, qp_hbm, h2s_hbm, dinv_hbm, b2_hbm,
              out_hbm, zcp_hbm,
              sidx_v, didx_v, zs, zd, ov, gsem, osem, s_in, s_out):
    cid = lax.axis_index("c")
    sid = lax.axis_index("s")
    wid = cid * NS + sid
    tbase = wid * EPT
    lanes = lax.iota(_i32, 16)
    pltpu.sync_copy(src_hbm.at[pl.ds(tbase, EPT)], sidx_v)
    pltpu.sync_copy(dst_hbm.at[pl.ds(tbase, EPT)], didx_v)

    pltpu.sync_copy(b2_hbm, ov[1].at[pl.ds(0, DO)])
    b2lo = ov[1][pl.ds(0, 16)]
    b2hi = ov[1][pl.ds(16, 16)]

    @pl.when(sid < 10)
    def _():
        def start_stage(c):
            h = (c % 2) * 200
            r0 = sid * 1000 + c * 200
            b = 4 * (c % 2)
            return (
                pltpu.async_copy(qp_hbm.at[pl.ds(r0, 200)],
                                 zs[0].at[pl.ds(h, 200)], s_in[b]),
                pltpu.async_copy(qp_hbm.at[pl.ds(NN + r0, 200)],
                                 zs[1].at[pl.ds(h, 200)], s_in[b + 1]),
                pltpu.async_copy(h2s_hbm.at[pl.ds(r0, 200)],
                                 zs[2].at[pl.ds(h, 200)], s_in[b + 2]),
                pltpu.async_copy(dinv_hbm.at[pl.ds(r0, 200)],
                                 ov[0].at[pl.ds(h, 200)], s_in[b + 3]),
            )

        ind = [None] * 5
        outd = [None] * 5
        ind[0] = start_stage(0)
        for c in range(5):
            h = (c % 2) * 200
            r0 = sid * 1000 + c * 200
            if c + 1 < 5:
                ind[c + 1] = start_stage(c + 1)
            for dsc in ind[c]:
                dsc.wait()
            if c >= 2:
                outd[c - 2].wait()

            def zrow(i, carry):
                dv = ov[0][pl.ds(h + i, 16)][0]
                lo = (zs[0][h + i, pl.ds(0, 16)] + zs[1][h + i, pl.ds(0, 16)]
                      + zs[2][h + i, pl.ds(0, 16)]) * dv + b2lo
                hi = (zs[0][h + i, pl.ds(16, 16)] + zs[1][h + i, pl.ds(16, 16)]
                      + zs[2][h + i, pl.ds(16, 16)]) * dv + b2hi
                zd[0][h + i, pl.ds(0, 16)] = lo
                zd[0][h + i, pl.ds(16, 16)] = hi
                return carry

            lax.fori_loop(0, 200, zrow, 0)
            outd[c] = pltpu.async_copy(zd[0].at[pl.ds(h, 200)],
                                       zcp_hbm.at[pl.ds(cid * NN + r0, 200)],
                                       s_out[c % 2])
        outd[3].wait()
        outd[4].wait()

    def bias_idx(i, carry):
        o = 16 * i
        sidx_v[pl.ds(o, 16)] = sidx_v[pl.ds(o, 16)] + cid * NN
        didx_v[pl.ds(o, 16)] = didx_v[pl.ds(o, 16)] + cid * NN
        return carry

    lax.fori_loop(0, EPT // 16, bias_idx, 0)
    plsc.subcore_barrier()

    cols = [(lanes + d) & (DO - 1) for d in range(DO)]

    def compute(zs_v, zd_v, ov_v):
        def group(g, carry):
            rows = g * 16 + lanes
            accs = [jnp.zeros((16,), _f32) for _ in range(4)]
            for d in range(DO):
                col = cols[d]
                a = plsc.load_gather(zs_v, [rows, col])
                b = plsc.load_gather(zd_v, [rows, col])
                accs[d % 4] = accs[d % 4] + a * b
            acc = (accs[0] + accs[1]) + (accs[2] + accs[3])
            sg = 1.0 / (1.0 + jnp.exp(-acc))
            plsc.store_scatter(ov_v, [rows], sg)
            return carry
        lax.fori_loop(0, _DEC_B // 16, group, 0)

    def start_gather(k):
        p = k % 3
        e0 = k * _DEC_B
        src = zcp_hbm
        return (
            pltpu.async_copy(src.at[sidx_v.at[pl.ds(e0, _DEC_B)]],
                             zs[p], gsem[2 * p]),
            pltpu.async_copy(src.at[didx_v.at[pl.ds(e0, _DEC_B)]],
                             zd[p], gsem[2 * p + 1]),
        )

    gd = [None] * _DEC_NB
    od = [None] * _DEC_NB
    gd[0] = start_gather(0)
    gd[1] = start_gather(1)
    for k in range(_DEC_NB):
        p = k % 3
        if k + 2 < _DEC_NB:
            gd[k + 2] = start_gather(k + 2)
        gd[k][0].wait()
        gd[k][1].wait()
        if k >= 2:
            od[k - 2].wait()
        q = k % 2
        compute(zs[p], zd[p], ov[q])
        od[k] = pltpu.async_copy(ov[q].at[pl.ds(0, _DEC_B)],
                                 out_hbm.at[pl.ds(tbase + k * _DEC_B,
                                                  _DEC_B)], osem[q])
    od[_DEC_NB - 2].wait()
    od[_DEC_NB - 1].wait()


_dec_call = pl.kernel(
    _dec_body,
    out_type=(jax.ShapeDtypeStruct((NE,), _f32),
              jax.ShapeDtypeStruct((NC * NN, DO), _f32)),
    mesh=_MESH,
    compiler_params=_SC_PARAMS,
    scratch_types=[
        pltpu.VMEM((EPT,), _i32),
        pltpu.VMEM((EPT,), _i32),
        [pltpu.VMEM((_DEC_B, DO), _f32)] * 3,
        [pltpu.VMEM((_DEC_B, DO), _f32)] * 3,
        [pltpu.VMEM((_DEC_B + 16,), _f32)] * 2,
        [pltpu.SemaphoreType.DMA] * 6,
        [pltpu.SemaphoreType.DMA] * 2,
        [pltpu.SemaphoreType.DMA] * 8,
        [pltpu.SemaphoreType.DMA] * 2,
    ],
)


def _enc1_body(degp_ref, x_ref, w1_ref, dinv_ref, hs_ref):
    deg = degp_ref[:NN] + degp_ref[NN:] + 1.0
    dinv = lax.rsqrt(deg)
    dinv_ref[...] = dinv
    h = jnp.dot(x_ref[...], w1_ref[...], preferred_element_type=_f32)
    hs_ref[...] = h * dinv[:, None]


_enc1_call = pl.pallas_call(
    _enc1_body,
    out_shape=(jax.ShapeDtypeStruct((NN,), _f32),
               jax.ShapeDtypeStruct((NN, DH), _f32)),
)


def _enc2_body(p_ref, hs_ref, dinv_ref, b1_ref, w2_ref, h2s_ref):
    dinv = dinv_ref[...]
    h = jnp.maximum(
        dinv[:, None] * (p_ref[:NN, :] + p_ref[NN:, :] + hs_ref[...])
        + b1_ref[...], 0.0)
    h2 = jnp.dot(h, w2_ref[...], preferred_element_type=_f32)
    h2s_ref[...] = h2 * dinv[:, None]


_enc2_call = pl.pallas_call(
    _enc2_body,
    out_shape=jax.ShapeDtypeStruct((NN, DO), _f32),
)


def kernel(x, edge_index, W1, b1, W2, b2):
    src = edge_index[0].astype(_i32)
    dst = edge_index[1].astype(_i32)
    degp = _deg_call(dst)
    dinv, hs = _enc1_call(degp, x, W1)
    P = _agg_h(src, dst, hs)
    h2s = _enc2_call(P, hs, dinv, b1, W2)
    Q = _agg_o(src, dst, h2s)
    out, _ = _dec_call(src, dst, Q, h2s, dinv, b2)
    return out
